# baseline (speedup 1.0000x reference)
import jax
import jax.numpy as jnp
from jax.experimental import pallas as pl
from jax.experimental.pallas import tpu as pltpu

_EPS = 1e-4


def _reduce_kernel(y_ref, m_ref, acc_ref):
    i = pl.program_id(0)
    w = (m_ref[...] > 0).astype(jnp.float32)
    yb = y_ref[...]
    wy = yb * w
    s = jnp.sum(wy, axis=0)
    sq = jnp.sum(wy * yb, axis=0)
    n = jnp.sum(w)

    @pl.when(i == 0)
    def _():
        acc_ref[...] = jnp.zeros_like(acc_ref)

    acc_ref[0, :] += s
    acc_ref[1, :] += sq
    acc_ref[2, :] += n


def _apply_kernel(acc_ref, g_ref, b_ref, y_ref, m_ref, o_ref):
    s = acc_ref[0, :]
    sq = acc_ref[1, :]
    n = acc_ref[2, :]
    mean = s / n
    var = (sq - s * mean) / (n - 1.0)
    std = jnp.sqrt(var)
    scale = g_ref[0, :] / (std + _EPS)
    shift = b_ref[0, :] - mean * scale
    yb = y_ref[...]
    sel = m_ref[...] > 0
    o_ref[...] = jnp.where(sel, yb * scale + shift, yb)


def kernel(y, mask, gamma, beta):
    B, T, C = y.shape
    rows = B * T
    y2 = y.reshape(rows, C)
    m2 = mask.reshape(rows, 1)

    R = 512
    grid = rows // R

    acc = pl.pallas_call(
        _reduce_kernel,
        grid=(grid,),
        in_specs=[
            pl.BlockSpec((R, C), lambda i: (i, 0)),
            pl.BlockSpec((R, 1), lambda i: (i, 0)),
        ],
        out_specs=pl.BlockSpec((8, C), lambda i: (0, 0)),
        out_shape=jax.ShapeDtypeStruct((8, C), jnp.float32),
        compiler_params=pltpu.CompilerParams(
            dimension_semantics=("arbitrary",),
        ),
    )(y2, m2)

    out = pl.pallas_call(
        _apply_kernel,
        grid=(grid,),
        in_specs=[
            pl.BlockSpec((8, C), lambda i: (0, 0)),
            pl.BlockSpec((1, C), lambda i: (0, 0)),
            pl.BlockSpec((1, C), lambda i: (0, 0)),
            pl.BlockSpec((R, C), lambda i: (i, 0)),
            pl.BlockSpec((R, 1), lambda i: (i, 0)),
        ],
        out_specs=pl.BlockSpec((R, C), lambda i: (i, 0)),
        out_shape=jax.ShapeDtypeStruct((rows, C), jnp.float32),
        compiler_params=pltpu.CompilerParams(
            dimension_semantics=("parallel",),
        ),
    )(acc, gamma.reshape(1, C), beta.reshape(1, C), y2, m2)

    return out.reshape(B, T, C)


# speedup vs baseline: 1.1716x; 1.1716x over previous
import jax
import jax.numpy as jnp
from jax.experimental import pallas as pl
from jax.experimental.pallas import tpu as pltpu

_EPS = 1e-4


def _reduce_kernel(y_ref, m_ref, acc_ref):
    i = pl.program_id(0)
    w = (m_ref[...] > 0).astype(jnp.float32)
    yb = y_ref[...]
    wy = yb * w
    s = jnp.sum(wy, axis=0)
    sq = jnp.sum(wy * yb, axis=0)
    n = jnp.sum(w)

    @pl.when(i == 0)
    def _():
        acc_ref[...] = jnp.zeros_like(acc_ref)

    acc_ref[0, :] += s
    acc_ref[1, :] += sq
    acc_ref[2, :] += n


def _apply_kernel(acc_ref, g_ref, b_ref, y_ref, m_ref, o_ref):
    s = acc_ref[0, :]
    sq = acc_ref[1, :]
    n = acc_ref[2, :]
    mean = s / n
    var = (sq - s * mean) / (n - 1.0)
    std = jnp.sqrt(var)
    scale = g_ref[0, :] / (std + _EPS)
    shift = b_ref[0, :] - mean * scale
    yb = y_ref[...]
    sel = m_ref[...] > 0
    o_ref[...] = jnp.where(sel, yb * scale + shift, yb)


def kernel(y, mask, gamma, beta):
    B, T, C = y.shape
    rows = B * T
    y2 = y.reshape(rows, C)
    m2 = mask.reshape(rows, 1)

    R = 1024
    grid = rows // R

    acc = pl.pallas_call(
        _reduce_kernel,
        grid=(grid,),
        in_specs=[
            pl.BlockSpec((R, C), lambda i: (i, 0)),
            pl.BlockSpec((R, 1), lambda i: (i, 0)),
        ],
        out_specs=pl.BlockSpec((8, C), lambda i: (0, 0)),
        out_shape=jax.ShapeDtypeStruct((8, C), jnp.float32),
        compiler_params=pltpu.CompilerParams(
            dimension_semantics=("arbitrary",),
        ),
    )(y2, m2)

    out = pl.pallas_call(
        _apply_kernel,
        grid=(grid,),
        in_specs=[
            pl.BlockSpec((8, C), lambda i: (0, 0)),
            pl.BlockSpec((1, C), lambda i: (0, 0)),
            pl.BlockSpec((1, C), lambda i: (0, 0)),
            pl.BlockSpec((R, C), lambda i: (i, 0)),
            pl.BlockSpec((R, 1), lambda i: (i, 0)),
        ],
        out_specs=pl.BlockSpec((R, C), lambda i: (i, 0)),
        out_shape=jax.ShapeDtypeStruct((rows, C), jnp.float32),
        compiler_params=pltpu.CompilerParams(
            dimension_semantics=("parallel",),
        ),
    )(acc, gamma.reshape(1, C), beta.reshape(1, C), y2, m2)

    return out.reshape(B, T, C)


# speedup vs baseline: 1.2312x; 1.0509x over previous
import jax
import jax.numpy as jnp
from jax.experimental import pallas as pl
from jax.experimental.pallas import tpu as pltpu

_EPS = 1e-4


def _reduce_kernel(y_ref, m_ref, acc_ref):
    i = pl.program_id(0)
    w = (m_ref[...] > 0).astype(jnp.float32)
    yb = y_ref[...]
    wy = yb * w
    s = jnp.sum(wy, axis=0)
    sq = jnp.sum(wy * yb, axis=0)
    n = jnp.sum(w)

    @pl.when(i == 0)
    def _():
        acc_ref[...] = jnp.zeros_like(acc_ref)

    acc_ref[0, :] += s
    acc_ref[1, :] += sq
    acc_ref[2, :] += n


def _apply_kernel(acc_ref, g_ref, b_ref, y_ref, m_ref, o_ref):
    s = acc_ref[0, :]
    sq = acc_ref[1, :]
    n = acc_ref[2, :]
    mean = s / n
    var = (sq - s * mean) / (n - 1.0)
    std = jnp.sqrt(var)
    scale = g_ref[0, :] / (std + _EPS)
    shift = b_ref[0, :] - mean * scale
    yb = y_ref[...]
    sel = m_ref[...] > 0
    o_ref[...] = jnp.where(sel, yb * scale + shift, yb)


def kernel(y, mask, gamma, beta):
    B, T, C = y.shape
    rows = B * T
    y2 = y.reshape(rows, C)
    m2 = mask.reshape(rows, 1)

    R = 2048
    grid = rows // R

    acc = pl.pallas_call(
        _reduce_kernel,
        grid=(grid,),
        in_specs=[
            pl.BlockSpec((R, C), lambda i: (i, 0)),
            pl.BlockSpec((R, 1), lambda i: (i, 0)),
        ],
        out_specs=pl.BlockSpec((8, C), lambda i: (0, 0)),
        out_shape=jax.ShapeDtypeStruct((8, C), jnp.float32),
        compiler_params=pltpu.CompilerParams(
            dimension_semantics=("arbitrary",),
        ),
    )(y2, m2)

    out = pl.pallas_call(
        _apply_kernel,
        grid=(grid,),
        in_specs=[
            pl.BlockSpec((8, C), lambda i: (0, 0)),
            pl.BlockSpec((1, C), lambda i: (0, 0)),
            pl.BlockSpec((1, C), lambda i: (0, 0)),
            pl.BlockSpec((R, C), lambda i: (i, 0)),
            pl.BlockSpec((R, 1), lambda i: (i, 0)),
        ],
        out_specs=pl.BlockSpec((R, C), lambda i: (i, 0)),
        out_shape=jax.ShapeDtypeStruct((rows, C), jnp.float32),
        compiler_params=pltpu.CompilerParams(
            dimension_semantics=("parallel",),
        ),
    )(acc, gamma.reshape(1, C), beta.reshape(1, C), y2, m2)

    return out.reshape(B, T, C)


# speedup vs baseline: 1.2354x; 1.0035x over previous
import jax
import jax.numpy as jnp
from jax.experimental import pallas as pl
from jax.experimental.pallas import tpu as pltpu

_EPS = 1e-4


def _reduce_kernel(y_ref, m_ref, acc_ref):
    i = pl.program_id(0)
    w = (m_ref[...] > 0).astype(jnp.float32)
    yb = y_ref[...]
    zb = yb * yb
    dn = (((0,), (0,)), ((), ()))
    s = jax.lax.dot_general(w, yb, dn, preferred_element_type=jnp.float32)
    sq = jax.lax.dot_general(w, zb, dn, preferred_element_type=jnp.float32)
    n = jnp.sum(w)

    @pl.when(i == 0)
    def _():
        acc_ref[...] = jnp.zeros_like(acc_ref)

    acc_ref[0, :] += s[0, :]
    acc_ref[1, :] += sq[0, :]
    acc_ref[2, :] += n


def _apply_kernel(acc_ref, g_ref, b_ref, y_ref, m_ref, o_ref):
    s = acc_ref[0, :]
    sq = acc_ref[1, :]
    n = acc_ref[2, :]
    mean = s / n
    var = (sq - s * mean) / (n - 1.0)
    std = jnp.sqrt(var)
    scale = g_ref[0, :] / (std + _EPS)
    shift = b_ref[0, :] - mean * scale
    yb = y_ref[...]
    sel = m_ref[...] > 0
    o_ref[...] = jnp.where(sel, yb * scale + shift, yb)


def kernel(y, mask, gamma, beta):
    B, T, C = y.shape
    rows = B * T
    y2 = y.reshape(rows, C)
    m2 = mask.reshape(rows, 1)

    R = 2048
    grid = rows // R

    acc = pl.pallas_call(
        _reduce_kernel,
        grid=(grid,),
        in_specs=[
            pl.BlockSpec((R, C), lambda i: (i, 0)),
            pl.BlockSpec((R, 1), lambda i: (i, 0)),
        ],
        out_specs=pl.BlockSpec((8, C), lambda i: (0, 0)),
        out_shape=jax.ShapeDtypeStruct((8, C), jnp.float32),
        compiler_params=pltpu.CompilerParams(
            dimension_semantics=("arbitrary",),
            vmem_limit_bytes=120 * 1024 * 1024,
        ),
    )(y2, m2)

    out = pl.pallas_call(
        _apply_kernel,
        grid=(grid,),
        in_specs=[
            pl.BlockSpec((8, C), lambda i: (0, 0)),
            pl.BlockSpec((1, C), lambda i: (0, 0)),
            pl.BlockSpec((1, C), lambda i: (0, 0)),
            pl.BlockSpec((R, C), lambda i: (i, 0)),
            pl.BlockSpec((R, 1), lambda i: (i, 0)),
        ],
        out_specs=pl.BlockSpec((R, C), lambda i: (i, 0)),
        out_shape=jax.ShapeDtypeStruct((rows, C), jnp.float32),
        compiler_params=pltpu.CompilerParams(
            dimension_semantics=("parallel",),
            vmem_limit_bytes=120 * 1024 * 1024,
        ),
    )(acc, gamma.reshape(1, C), beta.reshape(1, C), y2, m2)

    return out.reshape(B, T, C)
